# Initial kernel scaffold; baseline (speedup 1.0000x reference)
import jax
import jax.numpy as jnp
from jax.experimental import pallas as pl


def kernel(rois, gt_boxes):
    raise NotImplementedError("write your pallas kernel here")


# speedup vs baseline: 1.1262x; 1.1262x over previous
import jax
import jax.numpy as jnp
from jax import lax
from jax.experimental import pallas as pl

_R = 20000
_P = 20480
_ROWS = 160
_LANES = 128
_NGT = 50
_NIMG = 2
_PER_IMG = 256
_FG = 64
_BG = 192
_NCOL = 324
_NCOLP = 384
_BIG = 2 ** 30


def _lane(v, i):
    return jnp.sum(v[:, i:i + 1])


def _kernel_body(rt_ref, rois8_ref, gt_ref, oroi_ref, olab_ref, obt_ref, obw_ref):
    bidxg = rt_ref[0]
    x1g = rt_ref[1]
    y1g = rt_ref[2]
    x2g = rt_ref[3]
    y2g = rt_ref[4]

    idxg = (lax.broadcasted_iota(jnp.int32, (_ROWS, _LANES), 0) * _LANES
            + lax.broadcasted_iota(jnp.int32, (_ROWS, _LANES), 1))
    area_a = (x2g - x1g + 1.0) * (y2g - y1g + 1.0)
    ci = lax.broadcasted_iota(jnp.int32, (1, _NCOLP), 1)
    zrow = jnp.zeros((1, _NCOLP), jnp.float32)

    for b in range(_NIMG):
        def gt_step(g, carry):
            m, am = carry
            gv = gt_ref[pl.ds(b * _NGT + g, 1), :]
            bx1 = _lane(gv, 0)
            by1 = _lane(gv, 1)
            bx2 = _lane(gv, 2)
            by2 = _lane(gv, 3)
            iw = jnp.maximum(jnp.minimum(x2g, bx2) - jnp.maximum(x1g, bx1) + 1.0, 0.0)
            ih = jnp.maximum(jnp.minimum(y2g, by2) - jnp.maximum(y1g, by1) + 1.0, 0.0)
            inter = iw * ih
            area_b = (bx2 - bx1 + 1.0) * (by2 - by1 + 1.0)
            union = area_a + area_b - inter
            ov = inter / union
            better = ov > m
            return jnp.where(better, ov, m), jnp.where(better, g, am)

        m, am = lax.fori_loop(
            0, _NGT, gt_step,
            (jnp.full((_ROWS, _LANES), -jnp.inf, jnp.float32),
             jnp.zeros((_ROWS, _LANES), jnp.int32)))

        in_img = bidxg == jnp.float32(b)
        fgs = jnp.where(in_img & (m >= 0.5), m, -1.0)
        bgs = jnp.where(in_img & (m < 0.5), m, -1.0)

        def fg_body(i, s):
            val = jnp.max(s)
            sel = jnp.min(jnp.where(s == val, idxg, _BIG))
            s = jnp.where(idxg == sel, -2.0, s)
            hit = idxg == sel
            rv = rois8_ref[pl.ds(sel, 1), :]
            ga = jnp.min(jnp.where(hit, am, _BIG))
            gv = gt_ref[pl.ds(b * _NGT + ga, 1), :]
            valid = val > 0.0
            fgf = valid.astype(jnp.float32)
            label = jnp.where(valid, _lane(gv, 4), 0.0)
            ex_w = _lane(rv, 3) - _lane(rv, 1) + 1.0
            ex_h = _lane(rv, 4) - _lane(rv, 2) + 1.0
            ex_cx = _lane(rv, 1) + 0.5 * ex_w
            ex_cy = _lane(rv, 2) + 0.5 * ex_h
            gt_w = _lane(gv, 2) - _lane(gv, 0) + 1.0
            gt_h = _lane(gv, 3) - _lane(gv, 1) + 1.0
            gt_cx = _lane(gv, 0) + 0.5 * gt_w
            gt_cy = _lane(gv, 1) + 0.5 * gt_h
            dx = (gt_cx - ex_cx) / ex_w
            dy = (gt_cy - ex_cy) / ex_h
            dw = jnp.log(gt_w / ex_w)
            dh = jnp.log(gt_h / ex_h)
            row = b * _PER_IMG + i
            oroi_ref[pl.ds(row, 1), :] = rv
            olab_ref[pl.ds(row, 1), :] = jnp.broadcast_to(label, (1, 8))
            cls = label.astype(jnp.int32)
            maskc = (ci >> 2) == cls
            j = ci & 3
            tsel = jnp.where(j == 0, dx,
                             jnp.where(j == 1, dy, jnp.where(j == 2, dw, dh)))
            obt_ref[pl.ds(row, 1), :] = jnp.where(maskc, tsel * fgf, 0.0)
            obw_ref[pl.ds(row, 1), :] = jnp.where(maskc, fgf, 0.0)
            return s

        def bg_body(i, s):
            val = jnp.max(s)
            sel = jnp.min(jnp.where(s == val, idxg, _BIG))
            s = jnp.where(idxg == sel, -2.0, s)
            rv = rois8_ref[pl.ds(sel, 1), :]
            row = b * _PER_IMG + _FG + i
            oroi_ref[pl.ds(row, 1), :] = rv
            olab_ref[pl.ds(row, 1), :] = jnp.zeros((1, 8), jnp.float32)
            obt_ref[pl.ds(row, 1), :] = zrow
            obw_ref[pl.ds(row, 1), :] = zrow
            return s

        lax.fori_loop(0, _FG, fg_body, fgs)
        lax.fori_loop(0, _BG, bg_body, bgs)


def _build_call(interpret=False):
    return pl.pallas_call(
        _kernel_body,
        out_shape=[
            jax.ShapeDtypeStruct((_NIMG * _PER_IMG, 8), jnp.float32),
            jax.ShapeDtypeStruct((_NIMG * _PER_IMG, 8), jnp.float32),
            jax.ShapeDtypeStruct((_NIMG * _PER_IMG, _NCOLP), jnp.float32),
            jax.ShapeDtypeStruct((_NIMG * _PER_IMG, _NCOLP), jnp.float32),
        ],
        interpret=interpret,
    )


def kernel(rois, gt_boxes):
    rt = jnp.pad(rois.T, ((0, 0), (0, _P - _R)), constant_values=-1.0)
    rt = rt.reshape(5, _ROWS, _LANES)
    rois8 = jnp.pad(rois, ((0, 0), (0, 3)))
    gt2 = jnp.pad(gt_boxes.reshape(_NIMG * _NGT, 5), ((0, 0), (0, 3)))
    oroi, olab, obt, obw = _build_call()(rt, rois8, gt2)
    return oroi[:, :5], olab[:, 0], obt[:, :_NCOL], obw[:, :_NCOL]


# speedup vs baseline: 1.5258x; 1.3548x over previous
import jax
import jax.numpy as jnp
from jax import lax
from jax.experimental import pallas as pl

_R = 20000
_P = 20480
_ROWS = 160
_LANES = 128
_NGT = 50
_NIMG = 2
_PER_IMG = 256
_FG = 64
_BG = 192
_NCOL = 324
_NCOLP = 384
_BIG = 2 ** 30


def _lane(v, i):
    return jnp.sum(v[:, i:i + 1])


def _kernel_body(rt_ref, rois8_ref, gt_ref, oroi_ref, olab_ref, obt_ref, obw_ref):
    bidxg = rt_ref[0]
    x1g = rt_ref[1]
    y1g = rt_ref[2]
    x2g = rt_ref[3]
    y2g = rt_ref[4]

    idxg = (lax.broadcasted_iota(jnp.int32, (_ROWS, _LANES), 0) * _LANES
            + lax.broadcasted_iota(jnp.int32, (_ROWS, _LANES), 1))
    area_a = (x2g - x1g + 1.0) * (y2g - y1g + 1.0)
    ci = lax.broadcasted_iota(jnp.int32, (1, _NCOLP), 1)
    zrow = jnp.zeros((1, _NCOLP), jnp.float32)

    def one_iou(gv, m, am, g):
        bx1 = _lane(gv, 0)
        by1 = _lane(gv, 1)
        bx2 = _lane(gv, 2)
        by2 = _lane(gv, 3)
        iw = jnp.maximum(jnp.minimum(x2g, bx2) - jnp.maximum(x1g, bx1) + 1.0, 0.0)
        ih = jnp.maximum(jnp.minimum(y2g, by2) - jnp.maximum(y1g, by1) + 1.0, 0.0)
        inter = iw * ih
        area_b = (bx2 - bx1 + 1.0) * (by2 - by1 + 1.0)
        union = area_a + area_b - inter
        ov = inter / union
        better = ov > m
        return jnp.where(better, ov, m), jnp.where(better, g, am)

    def gt_step(g, carry):
        m0, am0, m1, am1 = carry
        gv0 = gt_ref[pl.ds(g, 1), :]
        gv1 = gt_ref[pl.ds(_NGT + g, 1), :]
        m0, am0 = one_iou(gv0, m0, am0, g)
        m1, am1 = one_iou(gv1, m1, am1, g)
        return m0, am0, m1, am1

    ninf = jnp.full((_ROWS, _LANES), -jnp.inf, jnp.float32)
    zidx = jnp.zeros((_ROWS, _LANES), jnp.int32)
    m0, am0, m1, am1 = lax.fori_loop(0, _NGT, gt_step, (ninf, zidx, ninf, zidx))

    def scores(b, m):
        in_img = bidxg == jnp.float32(b)
        fgs = jnp.where(in_img & (m >= 0.5), m, -1.0)
        bgs = jnp.where(in_img & (m < 0.5), m, -1.0)
        return fgs, bgs

    fgs0, bgs0 = scores(0, m0)
    fgs1, bgs1 = scores(1, m1)

    def extract(s):
        val = jnp.max(s)
        sel = jnp.min(jnp.where(s == val, idxg, _BIG))
        hit = idxg == sel
        return val, sel, hit, jnp.where(hit, -2.0, s)

    def fg_store(b, i, am, val, sel, hit):
        rv = rois8_ref[pl.ds(sel, 1), :]
        ga = jnp.min(jnp.where(hit, am, _BIG))
        gv = gt_ref[pl.ds(b * _NGT + ga, 1), :]
        valid = val > 0.0
        fgf = valid.astype(jnp.float32)
        label = jnp.where(valid, _lane(gv, 4), 0.0)
        ex_w = _lane(rv, 3) - _lane(rv, 1) + 1.0
        ex_h = _lane(rv, 4) - _lane(rv, 2) + 1.0
        ex_cx = _lane(rv, 1) + 0.5 * ex_w
        ex_cy = _lane(rv, 2) + 0.5 * ex_h
        gt_w = _lane(gv, 2) - _lane(gv, 0) + 1.0
        gt_h = _lane(gv, 3) - _lane(gv, 1) + 1.0
        gt_cx = _lane(gv, 0) + 0.5 * gt_w
        gt_cy = _lane(gv, 1) + 0.5 * gt_h
        dx = (gt_cx - ex_cx) / ex_w
        dy = (gt_cy - ex_cy) / ex_h
        dw = jnp.log(gt_w / ex_w)
        dh = jnp.log(gt_h / ex_h)
        row = b * _PER_IMG + i
        oroi_ref[pl.ds(row, 1), :] = rv
        olab_ref[pl.ds(row, 1), :] = jnp.broadcast_to(label, (1, 8))
        cls = label.astype(jnp.int32)
        maskc = (ci >> 2) == cls
        j = ci & 3
        tsel = jnp.where(j == 0, dx,
                         jnp.where(j == 1, dy, jnp.where(j == 2, dw, dh)))
        obt_ref[pl.ds(row, 1), :] = jnp.where(maskc, tsel * fgf, 0.0)
        obw_ref[pl.ds(row, 1), :] = jnp.where(maskc, fgf, 0.0)

    def bg_store(b, i, sel):
        rv = rois8_ref[pl.ds(sel, 1), :]
        row = b * _PER_IMG + _FG + i
        oroi_ref[pl.ds(row, 1), :] = rv
        olab_ref[pl.ds(row, 1), :] = jnp.zeros((1, 8), jnp.float32)
        obt_ref[pl.ds(row, 1), :] = zrow
        obw_ref[pl.ds(row, 1), :] = zrow

    def body1(i, carry):
        f0, g0, f1, g1 = carry
        v0, s0, h0, f0 = extract(f0)
        v1, s1, h1, f1 = extract(f1)
        vb0, sb0, hb0, g0 = extract(g0)
        vb1, sb1, hb1, g1 = extract(g1)
        fg_store(0, i, am0, v0, s0, h0)
        fg_store(1, i, am1, v1, s1, h1)
        bg_store(0, i, sb0)
        bg_store(1, i, sb1)
        return f0, g0, f1, g1

    _, g0, _, g1 = lax.fori_loop(0, _FG, body1, (fgs0, bgs0, fgs1, bgs1))

    def body2(i, carry):
        g0, g1 = carry
        vb0, sb0, hb0, g0 = extract(g0)
        vb1, sb1, hb1, g1 = extract(g1)
        bg_store(0, i, sb0)
        bg_store(1, i, sb1)
        return g0, g1

    lax.fori_loop(_FG, _BG, body2, (g0, g1))


def _build_call(interpret=False):
    return pl.pallas_call(
        _kernel_body,
        out_shape=[
            jax.ShapeDtypeStruct((_NIMG * _PER_IMG, 8), jnp.float32),
            jax.ShapeDtypeStruct((_NIMG * _PER_IMG, 8), jnp.float32),
            jax.ShapeDtypeStruct((_NIMG * _PER_IMG, _NCOLP), jnp.float32),
            jax.ShapeDtypeStruct((_NIMG * _PER_IMG, _NCOLP), jnp.float32),
        ],
        interpret=interpret,
    )


def kernel(rois, gt_boxes):
    rt = jnp.pad(rois.T, ((0, 0), (0, _P - _R)), constant_values=-1.0)
    rt = rt.reshape(5, _ROWS, _LANES)
    rois8 = jnp.pad(rois, ((0, 0), (0, 3)))
    gt2 = jnp.pad(gt_boxes.reshape(_NIMG * _NGT, 5), ((0, 0), (0, 3)))
    oroi, olab, obt, obw = _build_call()(rt, rois8, gt2)
    return oroi[:, :5], olab[:, 0], obt[:, :_NCOL], obw[:, :_NCOL]


# speedup vs baseline: 1.6827x; 1.1028x over previous
import jax
import jax.numpy as jnp
from jax import lax
from jax.experimental import pallas as pl
from jax.experimental.pallas import tpu as pltpu

_R = 20000
_P = 20480
_ROWS = 160
_LANES = 128
_NGT = 50
_NIMG = 2
_PER_IMG = 256
_FG = 64
_BG = 192
_NCOL = 324
_NCOLP = 384
_BIG = 2 ** 30


def _lane(v, i):
    return jnp.sum(v[:, i:i + 1])


def _kernel_body(rt_ref, rois8_ref, gt_ref, oroi_ref, olab_ref, obt_ref, obw_ref,
                 sf0_ref, sb0_ref, sf1_ref, sb1_ref, cmp0_ref, cmp1_ref):
    bidxg = rt_ref[0]
    x1g = rt_ref[1]
    y1g = rt_ref[2]
    x2g = rt_ref[3]
    y2g = rt_ref[4]

    idxg = (lax.broadcasted_iota(jnp.int32, (_ROWS, _LANES), 0) * _LANES
            + lax.broadcasted_iota(jnp.int32, (_ROWS, _LANES), 1))
    area_a = (x2g - x1g + 1.0) * (y2g - y1g + 1.0)
    ci = lax.broadcasted_iota(jnp.int32, (1, _NCOLP), 1)
    zrow = jnp.zeros((1, _NCOLP), jnp.float32)

    def one_iou(gv, m, am, g):
        bx1 = _lane(gv, 0)
        by1 = _lane(gv, 1)
        bx2 = _lane(gv, 2)
        by2 = _lane(gv, 3)
        iw = jnp.maximum(jnp.minimum(x2g, bx2) - jnp.maximum(x1g, bx1) + 1.0, 0.0)
        ih = jnp.maximum(jnp.minimum(y2g, by2) - jnp.maximum(y1g, by1) + 1.0, 0.0)
        inter = iw * ih
        area_b = (bx2 - bx1 + 1.0) * (by2 - by1 + 1.0)
        union = area_a + area_b - inter
        ov = inter / union
        better = ov > m
        return jnp.where(better, ov, m), jnp.where(better, g, am)

    def gt_step(g, carry):
        m0, am0, m1, am1 = carry
        gv0 = gt_ref[pl.ds(g, 1), :]
        gv1 = gt_ref[pl.ds(_NGT + g, 1), :]
        m0, am0 = one_iou(gv0, m0, am0, g)
        m1, am1 = one_iou(gv1, m1, am1, g)
        return m0, am0, m1, am1

    ninf = jnp.full((_ROWS, _LANES), -jnp.inf, jnp.float32)
    zidx = jnp.zeros((_ROWS, _LANES), jnp.int32)
    m0, am0, m1, am1 = lax.fori_loop(0, _NGT, gt_step, (ninf, zidx, ninf, zidx))

    def scores(b, m):
        in_img = bidxg == jnp.float32(b)
        fgs = jnp.where(in_img & (m >= 0.5), m, -1.0)
        bgs = jnp.where(in_img & (m < 0.5), m, -1.0)
        return fgs, bgs

    fgs0, bgs0 = scores(0, m0)
    fgs1, bgs1 = scores(1, m1)

    cmp0_ref[...] = idxg * 64 + am0
    cmp1_ref[...] = idxg * 64 + am1
    sf0_ref[...] = fgs0
    sb0_ref[...] = bgs0
    sf1_ref[...] = fgs1
    sb1_ref[...] = bgs1

    def pick(va, ca, vb, cb):
        take = (va > vb) | ((va == vb) & (ca < cb))
        return jnp.where(take, va, vb), jnp.where(take, ca, cb)

    def blk_tree(v, c):
        v, c = pick(v[0:4], c[0:4], v[4:8], c[4:8])
        v, c = pick(v[0:2], c[0:2], v[2:4], c[2:4])
        v, c = pick(v[0:1], c[0:1], v[1:2], c[1:2])
        return v, c

    def build_summary(s, comp):
        vs, cs = [], []
        for k in range(_ROWS // 8):
            v, c = blk_tree(s[8 * k:8 * k + 8], comp[8 * k:8 * k + 8])
            vs.append(v)
            cs.append(c)
        return jnp.concatenate(vs, 0), jnp.concatenate(cs, 0)

    sub8 = lax.broadcasted_iota(jnp.int32, (8, _LANES), 0)
    lane8 = lax.broadcasted_iota(jnp.int32, (8, _LANES), 1)
    blkpos = sub8 * _LANES + lane8
    rowiota20 = lax.broadcasted_iota(jnp.int32, (_ROWS // 8, _LANES), 0)

    def extract(carry, s_ref, comp_ref):
        bv, bc = carry
        val = jnp.max(bv)
        selc = jnp.min(jnp.where(bv == val, bc, _BIG))
        lin = selc >> 6
        ga = selc & 63
        blkbase = (lin >> 10) << 3
        sblk = s_ref[pl.ds(blkbase, 8), :]
        sblk = jnp.where(blkpos == (lin & 1023), -2.0, sblk)
        s_ref[pl.ds(blkbase, 8), :] = sblk
        cblk = comp_ref[pl.ds(blkbase, 8), :]
        v, c = blk_tree(sblk, cblk)
        mrow = rowiota20 == (blkbase >> 3)
        bv = jnp.where(mrow, v, bv)
        bc = jnp.where(mrow, c, bc)
        return val, lin, ga, (bv, bc)

    def fg_store(b, i, val, sel, ga):
        rv = rois8_ref[pl.ds(sel, 1), :]
        gv = gt_ref[pl.ds(b * _NGT + ga, 1), :]
        valid = val > 0.0
        fgf = valid.astype(jnp.float32)
        label = jnp.where(valid, _lane(gv, 4), 0.0)
        ex_w = _lane(rv, 3) - _lane(rv, 1) + 1.0
        ex_h = _lane(rv, 4) - _lane(rv, 2) + 1.0
        ex_cx = _lane(rv, 1) + 0.5 * ex_w
        ex_cy = _lane(rv, 2) + 0.5 * ex_h
        gt_w = _lane(gv, 2) - _lane(gv, 0) + 1.0
        gt_h = _lane(gv, 3) - _lane(gv, 1) + 1.0
        gt_cx = _lane(gv, 0) + 0.5 * gt_w
        gt_cy = _lane(gv, 1) + 0.5 * gt_h
        dx = (gt_cx - ex_cx) / ex_w
        dy = (gt_cy - ex_cy) / ex_h
        dw = jnp.log(gt_w / ex_w)
        dh = jnp.log(gt_h / ex_h)
        row = b * _PER_IMG + i
        oroi_ref[pl.ds(row, 1), :] = rv
        olab_ref[pl.ds(row, 1), :] = jnp.broadcast_to(label, (1, 8))
        cls = label.astype(jnp.int32)
        maskc = (ci >> 2) == cls
        j = ci & 3
        tsel = jnp.where(j == 0, dx,
                         jnp.where(j == 1, dy, jnp.where(j == 2, dw, dh)))
        obt_ref[pl.ds(row, 1), :] = jnp.where(maskc, tsel * fgf, 0.0)
        obw_ref[pl.ds(row, 1), :] = jnp.where(maskc, fgf, 0.0)

    def bg_store(b, i, sel):
        rv = rois8_ref[pl.ds(sel, 1), :]
        row = b * _PER_IMG + _FG + i
        oroi_ref[pl.ds(row, 1), :] = rv
        olab_ref[pl.ds(row, 1), :] = jnp.zeros((1, 8), jnp.float32)
        obt_ref[pl.ds(row, 1), :] = zrow
        obw_ref[pl.ds(row, 1), :] = zrow

    sum_f0 = build_summary(fgs0, cmp0_ref[...])
    sum_b0 = build_summary(bgs0, cmp0_ref[...])
    sum_f1 = build_summary(fgs1, cmp1_ref[...])
    sum_b1 = build_summary(bgs1, cmp1_ref[...])

    def body1(i, carry):
        f0, g0, f1, g1 = carry
        v0, s0, a0, f0 = extract(f0, sf0_ref, cmp0_ref)
        v1, s1, a1, f1 = extract(f1, sf1_ref, cmp1_ref)
        vb0, sb0, _, g0 = extract(g0, sb0_ref, cmp0_ref)
        vb1, sb1, _, g1 = extract(g1, sb1_ref, cmp1_ref)
        fg_store(0, i, v0, s0, a0)
        fg_store(1, i, v1, s1, a1)
        bg_store(0, i, sb0)
        bg_store(1, i, sb1)
        return f0, g0, f1, g1

    _, g0, _, g1 = lax.fori_loop(0, _FG, body1, (sum_f0, sum_b0, sum_f1, sum_b1))

    def body2(i, carry):
        g0, g1 = carry
        vb0, sb0, _, g0 = extract(g0, sb0_ref, cmp0_ref)
        vb1, sb1, _, g1 = extract(g1, sb1_ref, cmp1_ref)
        bg_store(0, i, sb0)
        bg_store(1, i, sb1)
        return g0, g1

    lax.fori_loop(_FG, _BG, body2, (g0, g1))


def _build_call(interpret=False):
    return pl.pallas_call(
        _kernel_body,
        out_shape=[
            jax.ShapeDtypeStruct((_NIMG * _PER_IMG, 8), jnp.float32),
            jax.ShapeDtypeStruct((_NIMG * _PER_IMG, 8), jnp.float32),
            jax.ShapeDtypeStruct((_NIMG * _PER_IMG, _NCOLP), jnp.float32),
            jax.ShapeDtypeStruct((_NIMG * _PER_IMG, _NCOLP), jnp.float32),
        ],
        scratch_shapes=[
            pltpu.VMEM((_ROWS, _LANES), jnp.float32),
            pltpu.VMEM((_ROWS, _LANES), jnp.float32),
            pltpu.VMEM((_ROWS, _LANES), jnp.float32),
            pltpu.VMEM((_ROWS, _LANES), jnp.float32),
            pltpu.VMEM((_ROWS, _LANES), jnp.int32),
            pltpu.VMEM((_ROWS, _LANES), jnp.int32),
        ],
        interpret=interpret,
    )


def kernel(rois, gt_boxes):
    rt = jnp.pad(rois.T, ((0, 0), (0, _P - _R)), constant_values=-1.0)
    rt = rt.reshape(5, _ROWS, _LANES)
    rois8 = jnp.pad(rois, ((0, 0), (0, 3)))
    gt2 = jnp.pad(gt_boxes.reshape(_NIMG * _NGT, 5), ((0, 0), (0, 3)))
    oroi, olab, obt, obw = _build_call()(rt, rois8, gt2)
    return oroi[:, :5], olab[:, 0], obt[:, :_NCOL], obw[:, :_NCOL]


# speedup vs baseline: 1.7257x; 1.0255x over previous
import jax
import jax.numpy as jnp
from jax import lax
from jax.experimental import pallas as pl
from jax.experimental.pallas import tpu as pltpu

_R = 20000
_P = 20480
_ROWS = 160
_LANES = 128
_NGT = 50
_NIMG = 2
_PER_IMG = 256
_FG = 64
_BG = 192
_NCOL = 324
_NCOLP = 384
_BIG = 2 ** 30


def _lane(v, i):
    return jnp.sum(v[:, i:i + 1])


def _kernel_body(rt_ref, rois8_ref, gt_ref, oroi_ref, olab_ref, obt_ref, obw_ref,
                 sf0_ref, sb0_ref, sf1_ref, sb1_ref, cmp0_ref, cmp1_ref):
    bidxg = rt_ref[0]
    x1g = rt_ref[1]
    y1g = rt_ref[2]
    x2g = rt_ref[3]
    y2g = rt_ref[4]

    idxg = (lax.broadcasted_iota(jnp.int32, (_ROWS, _LANES), 0) * _LANES
            + lax.broadcasted_iota(jnp.int32, (_ROWS, _LANES), 1))
    area_a = (x2g - x1g + 1.0) * (y2g - y1g + 1.0)
    ci = lax.broadcasted_iota(jnp.int32, (1, _NCOLP), 1)
    zrow = jnp.zeros((1, _NCOLP), jnp.float32)

    def one_iou(gv, m, am, g):
        bx1 = gv[:, 0:1]
        by1 = gv[:, 1:2]
        bx2 = gv[:, 2:3]
        by2 = gv[:, 3:4]
        iw = jnp.maximum(jnp.minimum(x2g, bx2) - jnp.maximum(x1g, bx1) + 1.0, 0.0)
        ih = jnp.maximum(jnp.minimum(y2g, by2) - jnp.maximum(y1g, by1) + 1.0, 0.0)
        inter = iw * ih
        area_b = (bx2 - bx1 + 1.0) * (by2 - by1 + 1.0)
        union = area_a + area_b - inter
        ov = inter / union
        better = ov > m
        return jnp.where(better, ov, m), jnp.where(better, g, am)

    def gt_step(g, carry):
        m0, am0, m1, am1 = carry
        gv0 = gt_ref[pl.ds(g, 1), :]
        gv1 = gt_ref[pl.ds(_NGT + g, 1), :]
        m0, am0 = one_iou(gv0, m0, am0, g)
        m1, am1 = one_iou(gv1, m1, am1, g)
        return m0, am0, m1, am1

    ninf = jnp.full((_ROWS, _LANES), -jnp.inf, jnp.float32)
    zidx = jnp.zeros((_ROWS, _LANES), jnp.int32)
    m0, am0, m1, am1 = lax.fori_loop(0, _NGT, gt_step, (ninf, zidx, ninf, zidx))

    def scores(b, m):
        in_img = bidxg == jnp.float32(b)
        fgs = jnp.where(in_img & (m >= 0.5), m, -1.0)
        bgs = jnp.where(in_img & (m < 0.5), m, -1.0)
        return fgs, bgs

    fgs0, bgs0 = scores(0, m0)
    fgs1, bgs1 = scores(1, m1)

    cmp0_ref[...] = idxg * 64 + am0
    cmp1_ref[...] = idxg * 64 + am1
    sf0_ref[...] = fgs0
    sb0_ref[...] = bgs0
    sf1_ref[...] = fgs1
    sb1_ref[...] = bgs1

    def pick(va, ca, vb, cb):
        take = (va > vb) | ((va == vb) & (ca < cb))
        return jnp.where(take, va, vb), jnp.where(take, ca, cb)

    def blk_tree(v, c):
        v, c = pick(v[0:4], c[0:4], v[4:8], c[4:8])
        v, c = pick(v[0:2], c[0:2], v[2:4], c[2:4])
        v, c = pick(v[0:1], c[0:1], v[1:2], c[1:2])
        return v, c

    def build_summary(s, comp):
        vs, cs = [], []
        for k in range(_ROWS // 8):
            v, c = blk_tree(s[8 * k:8 * k + 8], comp[8 * k:8 * k + 8])
            vs.append(v)
            cs.append(c)
        return jnp.concatenate(vs, 0), jnp.concatenate(cs, 0)

    sub8 = lax.broadcasted_iota(jnp.int32, (8, _LANES), 0)
    lane8 = lax.broadcasted_iota(jnp.int32, (8, _LANES), 1)
    blkpos = sub8 * _LANES + lane8
    rowiota20 = lax.broadcasted_iota(jnp.int32, (_ROWS // 8, _LANES), 0)

    def extract(carry, s_ref, comp_ref):
        bv, bc = carry
        val = jnp.max(bv)
        selc = jnp.min(jnp.where(bv == val, bc, _BIG))
        lin = selc >> 6
        ga = selc & 63
        blkbase = (lin >> 10) << 3
        sblk = s_ref[pl.ds(blkbase, 8), :]
        sblk = jnp.where(blkpos == (lin & 1023), -2.0, sblk)
        s_ref[pl.ds(blkbase, 8), :] = sblk
        cblk = comp_ref[pl.ds(blkbase, 8), :]
        v, c = blk_tree(sblk, cblk)
        mrow = rowiota20 == (blkbase >> 3)
        bv = jnp.where(mrow, v, bv)
        bc = jnp.where(mrow, c, bc)
        return val, lin, ga, (bv, bc)

    lane8r = lax.broadcasted_iota(jnp.int32, (1, 8), 1)

    def fg_store(b, i, val, sel, ga):
        rv = rois8_ref[pl.ds(sel, 1), :]
        rvx = jnp.where(lane8r == 5, val,
                        jnp.where(lane8r == 6, ga.astype(jnp.float32), rv))
        oroi_ref[pl.ds(b * _PER_IMG + i, 1), :] = rvx

    def bg_store(b, i, sel):
        rv = rois8_ref[pl.ds(sel, 1), :]
        oroi_ref[pl.ds(b * _PER_IMG + _FG + i, 1), :] = rv

    sum_f0 = build_summary(fgs0, cmp0_ref[...])
    sum_b0 = build_summary(bgs0, cmp0_ref[...])
    sum_f1 = build_summary(fgs1, cmp1_ref[...])
    sum_b1 = build_summary(bgs1, cmp1_ref[...])

    olab_ref[...] = jnp.zeros((_NIMG * _PER_IMG, 8), jnp.float32)
    obt_ref[...] = jnp.zeros((_NIMG * _PER_IMG, _NCOLP), jnp.float32)
    obw_ref[...] = jnp.zeros((_NIMG * _PER_IMG, _NCOLP), jnp.float32)

    def body1(i, carry):
        f0, g0, f1, g1 = carry
        v0, s0, a0, f0 = extract(f0, sf0_ref, cmp0_ref)
        v1, s1, a1, f1 = extract(f1, sf1_ref, cmp1_ref)
        vb0, sb0, _, g0 = extract(g0, sb0_ref, cmp0_ref)
        vb1, sb1, _, g1 = extract(g1, sb1_ref, cmp1_ref)
        fg_store(0, i, v0, s0, a0)
        fg_store(1, i, v1, s1, a1)
        bg_store(0, i, sb0)
        bg_store(1, i, sb1)
        return f0, g0, f1, g1

    _, g0, _, g1 = lax.fori_loop(0, _FG, body1, (sum_f0, sum_b0, sum_f1, sum_b1))

    def body2(i, carry):
        g0, g1 = carry
        vb0, sb0, _, g0 = extract(g0, sb0_ref, cmp0_ref)
        vb1, sb1, _, g1 = extract(g1, sb1_ref, cmp1_ref)
        bg_store(0, i, sb0)
        bg_store(1, i, sb1)
        return g0, g1

    lax.fori_loop(_FG, _BG, body2, (g0, g1))

    F = jnp.concatenate(
        [oroi_ref[0:_FG, :], oroi_ref[_PER_IMG:_PER_IMG + _FG, :]], 0)
    val_c = F[:, 5:6]
    ga_c = F[:, 6:7].astype(jnp.int32)
    valid = val_c > 0.0
    fgf = valid.astype(jnp.float32)
    riota = lax.broadcasted_iota(jnp.int32, (2 * _FG, 1), 0)
    gidx = ga_c + jnp.where(riota < _FG, 0, _NGT)
    onehot = (gidx == lax.broadcasted_iota(
        jnp.int32, (2 * _FG, _NIMG * _NGT), 1)).astype(jnp.float32)
    G = jnp.dot(onehot, gt_ref[...], preferred_element_type=jnp.float32)
    label = jnp.where(valid, G[:, 4:5], 0.0)
    ex_w = F[:, 3:4] - F[:, 1:2] + 1.0
    ex_h = F[:, 4:5] - F[:, 2:3] + 1.0
    ex_cx = F[:, 1:2] + 0.5 * ex_w
    ex_cy = F[:, 2:3] + 0.5 * ex_h
    gt_w = G[:, 2:3] - G[:, 0:1] + 1.0
    gt_h = G[:, 3:4] - G[:, 1:2] + 1.0
    gt_cx = G[:, 0:1] + 0.5 * gt_w
    gt_cy = G[:, 1:2] + 0.5 * gt_h
    dx = (gt_cx - ex_cx) / ex_w
    dy = (gt_cy - ex_cy) / ex_h
    dw = jnp.log(gt_w / ex_w)
    dh = jnp.log(gt_h / ex_h)
    cif = lax.broadcasted_iota(jnp.int32, (2 * _FG, _NCOLP), 1)
    cls = label.astype(jnp.int32)
    maskc = (cif >> 2) == cls
    j = cif & 3
    tsel = jnp.where(j == 0, dx,
                     jnp.where(j == 1, dy, jnp.where(j == 2, dw, dh)))
    btF = jnp.where(maskc, tsel * fgf, 0.0)
    bwF = jnp.where(maskc, fgf * jnp.ones_like(tsel), 0.0)
    obt_ref[0:_FG, :] = btF[0:_FG]
    obt_ref[_PER_IMG:_PER_IMG + _FG, :] = btF[_FG:]
    obw_ref[0:_FG, :] = bwF[0:_FG]
    obw_ref[_PER_IMG:_PER_IMG + _FG, :] = bwF[_FG:]
    olab_ref[0:_FG, :] = jnp.broadcast_to(label[0:_FG], (_FG, 8))
    olab_ref[_PER_IMG:_PER_IMG + _FG, :] = jnp.broadcast_to(label[_FG:], (_FG, 8))


def _build_call(interpret=False):
    return pl.pallas_call(
        _kernel_body,
        out_shape=[
            jax.ShapeDtypeStruct((_NIMG * _PER_IMG, 8), jnp.float32),
            jax.ShapeDtypeStruct((_NIMG * _PER_IMG, 8), jnp.float32),
            jax.ShapeDtypeStruct((_NIMG * _PER_IMG, _NCOLP), jnp.float32),
            jax.ShapeDtypeStruct((_NIMG * _PER_IMG, _NCOLP), jnp.float32),
        ],
        scratch_shapes=[
            pltpu.VMEM((_ROWS, _LANES), jnp.float32),
            pltpu.VMEM((_ROWS, _LANES), jnp.float32),
            pltpu.VMEM((_ROWS, _LANES), jnp.float32),
            pltpu.VMEM((_ROWS, _LANES), jnp.float32),
            pltpu.VMEM((_ROWS, _LANES), jnp.int32),
            pltpu.VMEM((_ROWS, _LANES), jnp.int32),
        ],
        interpret=interpret,
    )


def kernel(rois, gt_boxes):
    rt = jnp.pad(rois.T, ((0, 0), (0, _P - _R)), constant_values=-1.0)
    rt = rt.reshape(5, _ROWS, _LANES)
    rois8 = jnp.pad(rois, ((0, 0), (0, 3)))
    gt2 = jnp.pad(gt_boxes.reshape(_NIMG * _NGT, 5), ((0, 0), (0, 3)))
    oroi, olab, obt, obw = _build_call()(rt, rois8, gt2)
    return oroi[:, :5], olab[:, 0], obt[:, :_NCOL], obw[:, :_NCOL]
